# baseline (device time: 40994 ns/iter reference)
import jax
import jax.numpy as jnp
from jax import lax
from jax.experimental import pallas as pl
from jax.experimental.pallas import tpu as pltpu

N_DEV = 32
EPS = 1e-5
N_CHUNK = 4


def kernel(x, Wp):
    b, hs, w, c = x.shape
    c_out = Wp.shape[1]
    n_global = hs * N_DEV * w
    ch = hs // N_CHUNK

    def body(x_ref, wp_ref, dummy_ref, out_hbm, ob, comm_ref,
             send_sems, recv_sems, store_sems):
        my_pos = lax.axis_index("i")

        barrier_sem = pltpu.get_barrier_semaphore()
        for d in range(1, N_DEV):
            peer = lax.rem(my_pos + d, N_DEV)
            pl.semaphore_signal(
                barrier_sem, inc=1,
                device_id=(peer,), device_id_type=pl.DeviceIdType.MESH,
            )

        xv = x_ref[...]
        comm_ref[0, 0] = jnp.sum(xv, axis=(1, 3))
        comm_ref[0, 1] = jnp.sum(xv * xv, axis=(1, 3))

        pl.semaphore_wait(barrier_sem, N_DEV - 1)

        rdmas = []
        for d in range(1, N_DEV):
            target = lax.rem(my_pos + d, N_DEV)
            rdma = pltpu.make_async_remote_copy(
                src_ref=comm_ref.at[0],
                dst_ref=comm_ref.at[d],
                send_sem=send_sems.at[d],
                recv_sem=recv_sems.at[d],
                device_id=(target,),
                device_id_type=pl.DeviceIdType.MESH,
            )
            rdma.start()
            rdmas.append(rdma)
        for rdma in rdmas:
            rdma.wait()

        stats = jnp.sum(comm_ref[...], axis=0)
        mean = stats[0] / n_global
        var = stats[1] / n_global - mean * mean
        inv = lax.rsqrt(var + EPS)
        mean_b = mean[:, None, :, None]
        inv_b = inv[:, None, :, None]

        stores = [None, None]
        for i in range(N_CHUNK):
            slot = i % 2
            if stores[slot] is not None:
                stores[slot].wait()
            xc = x_ref[:, pl.ds(i * ch, ch)]
            h = (xc - mean_b) * inv_b
            a = h * jax.nn.sigmoid(h)
            at = jnp.transpose(a, (0, 1, 3, 2))
            out2d = jnp.dot(
                at.reshape(b * ch * w, c), wp_ref[...],
                preferred_element_type=jnp.float32,
            )
            ob[slot] = out2d.reshape(b, ch, w, c_out)
            st = pltpu.make_async_copy(
                ob.at[slot],
                out_hbm.at[:, pl.ds(i * ch, ch)],
                store_sems.at[slot],
            )
            st.start()
            stores[slot] = st
        for st in stores:
            st.wait()

    xt = jnp.transpose(x, (0, 1, 3, 2))
    dummy = jnp.zeros((b, hs, w, c_out), jnp.float32)
    return pl.pallas_call(
        body,
        out_shape=jax.ShapeDtypeStruct((b, hs, w, c_out), jnp.float32),
        in_specs=[
            pl.BlockSpec(memory_space=pltpu.VMEM),
            pl.BlockSpec(memory_space=pltpu.VMEM),
            pl.BlockSpec(memory_space=pl.ANY),
        ],
        out_specs=pl.BlockSpec(memory_space=pl.ANY),
        input_output_aliases={2: 0},
        scratch_shapes=[
            pltpu.VMEM((2, b, hs // N_CHUNK, w, c_out), jnp.float32),
            pltpu.VMEM((N_DEV, 2, b, c), jnp.float32),
            pltpu.SemaphoreType.DMA((N_DEV,)),
            pltpu.SemaphoreType.DMA((N_DEV,)),
            pltpu.SemaphoreType.DMA((2,)),
        ],
        compiler_params=pltpu.CompilerParams(collective_id=0),
    )(xt, Wp, dummy)


# device time: 37567 ns/iter; 1.0912x vs baseline; 1.0912x over previous
import jax
import jax.numpy as jnp
from jax import lax
from jax.experimental import pallas as pl
from jax.experimental.pallas import tpu as pltpu

N_DEV = 32
EPS = 1e-5
N_CHUNK = 4


def kernel(x, Wp):
    b, hs, w, c = x.shape
    c_out = Wp.shape[1]
    n_global = hs * N_DEV * w
    ch = hs // N_CHUNK

    def body(x_ref, wp_ref, out_hbm, ob, comm_ref,
             send_sems, recv_sems, store_sems):
        my_pos = lax.axis_index("i")

        barrier_sem = pltpu.get_barrier_semaphore()
        for d in range(1, N_DEV):
            peer = lax.rem(my_pos + d, N_DEV)
            pl.semaphore_signal(
                barrier_sem, inc=1,
                device_id=(peer,), device_id_type=pl.DeviceIdType.MESH,
            )

        xv = x_ref[...]
        comm_ref[0, 0] = jnp.sum(xv, axis=(1, 3))
        comm_ref[0, 1] = jnp.sum(xv * xv, axis=(1, 3))

        pl.semaphore_wait(barrier_sem, N_DEV - 1)

        rdmas = []
        for d in range(1, N_DEV):
            target = lax.rem(my_pos + d, N_DEV)
            rdma = pltpu.make_async_remote_copy(
                src_ref=comm_ref.at[0],
                dst_ref=comm_ref.at[d],
                send_sem=send_sems.at[d],
                recv_sem=recv_sems.at[d],
                device_id=(target,),
                device_id_type=pl.DeviceIdType.MESH,
            )
            rdma.start()
            rdmas.append(rdma)
        for rdma in rdmas:
            rdma.wait()

        stats = jnp.sum(comm_ref[...], axis=0)
        mean = stats[0] / n_global
        var = stats[1] / n_global - mean * mean
        inv = lax.rsqrt(var + EPS)
        mean_b = mean[:, None, :, None]
        inv_b = inv[:, None, :, None]

        stores = [None, None]
        for i in range(N_CHUNK):
            slot = i % 2
            if stores[slot] is not None:
                stores[slot].wait()
            xc = x_ref[:, pl.ds(i * ch, ch)]
            h = (xc - mean_b) * inv_b
            a = h * jax.nn.sigmoid(h)
            at = jnp.transpose(a, (0, 1, 3, 2))
            out2d = jnp.dot(
                at.reshape(b * ch * w, c), wp_ref[...],
                preferred_element_type=jnp.float32,
            )
            ob[slot] = out2d.reshape(b, ch, w, c_out)
            st = pltpu.make_async_copy(
                ob.at[slot],
                out_hbm.at[:, pl.ds(i * ch, ch)],
                store_sems.at[slot],
            )
            st.start()
            stores[slot] = st
        for st in stores:
            st.wait()

    xt = jnp.transpose(x, (0, 1, 3, 2))
    return pl.pallas_call(
        body,
        out_shape=jax.ShapeDtypeStruct((b, hs, w, c_out), jnp.float32),
        in_specs=[
            pl.BlockSpec(memory_space=pltpu.VMEM),
            pl.BlockSpec(memory_space=pltpu.VMEM),
        ],
        out_specs=pl.BlockSpec(memory_space=pl.ANY),
        scratch_shapes=[
            pltpu.VMEM((2, b, hs // N_CHUNK, w, c_out), jnp.float32),
            pltpu.VMEM((N_DEV, 2, b, c), jnp.float32),
            pltpu.SemaphoreType.DMA((N_DEV,)),
            pltpu.SemaphoreType.DMA((N_DEV,)),
            pltpu.SemaphoreType.DMA((2,)),
        ],
        compiler_params=pltpu.CompilerParams(collective_id=0),
    )(xt, Wp)
